# baseline (speedup 1.0000x reference)
import functools

import jax
import jax.numpy as jnp
from jax import lax
from jax.experimental import pallas as pl
from jax.experimental.pallas import tpu as pltpu
from jax.experimental.pallas import tpu_sc as plsc

B, L, E = 4096, 200, 64
N = B * L
NC, NS = 2, 16
NW = NC * NS
W = N // NW
CH = 128
NCH = W // CH

_mesh = plsc.VectorSubcoreMesh(core_axis_name="c", subcore_axis_name="s")


@functools.partial(
    pl.kernel,
    out_type=jax.ShapeDtypeStruct((N, E), jnp.float32),
    mesh=_mesh,
    scratch_types=[
        pltpu.VMEM((NCH, CH), jnp.int32),
        pltpu.VMEM((CH, E), jnp.float32),
        pltpu.SemaphoreType.DMA,
    ],
    compiler_params=pltpu.CompilerParams(use_tc_tiling_on_sc=False),
)
def _emb(idx_hbm, table_hbm, out_hbm, idx_v, rows_v, gsem):
    wid = lax.axis_index("s") * NC + lax.axis_index("c")
    base_ch = wid * NCH
    pltpu.sync_copy(idx_hbm.at[pl.ds(base_ch, NCH)], idx_v)

    def body(j, carry):
        pltpu.async_copy(table_hbm.at[idx_v.at[j]], rows_v, gsem).wait()
        pltpu.sync_copy(rows_v, out_hbm.at[pl.ds((base_ch + j) * CH, CH)])
        return carry

    lax.fori_loop(0, NCH, body, 0)


def kernel(tok_ids, table):
    idx = tok_ids.reshape(NW * NCH, CH).astype(jnp.int32)
    out = _emb(idx, table)
    return out.reshape(B, L, E)


# speedup vs baseline: 1.1149x; 1.1149x over previous
import functools

import jax
import jax.numpy as jnp
from jax import lax
from jax.experimental import pallas as pl
from jax.experimental.pallas import tpu as pltpu
from jax.experimental.pallas import tpu_sc as plsc

B, L, E = 4096, 200, 64
N = B * L
NC, NS = 2, 16
NW = NC * NS
W = N // NW
CH = 128
NCH = W // CH
NBUF = 4
NG = NCH // NBUF

_mesh = plsc.VectorSubcoreMesh(core_axis_name="c", subcore_axis_name="s")


@functools.partial(
    pl.kernel,
    out_type=jax.ShapeDtypeStruct((N, E), jnp.float32),
    mesh=_mesh,
    scratch_types=[
        pltpu.VMEM((NCH, CH), jnp.int32),
        pltpu.VMEM((NBUF, CH, E), jnp.float32),
        [pltpu.SemaphoreType.DMA] * NBUF,
        [pltpu.SemaphoreType.DMA] * NBUF,
    ],
    compiler_params=pltpu.CompilerParams(use_tc_tiling_on_sc=False),
)
def _emb(idx_hbm, table_hbm, out_hbm, idx_v, rows_v, gsems, ssems):
    wid = lax.axis_index("s") * NC + lax.axis_index("c")
    base_ch = wid * NCH
    pltpu.sync_copy(idx_hbm.at[pl.ds(base_ch, NCH)], idx_v)

    def start_gather(j, b):
        pltpu.async_copy(table_hbm.at[idx_v.at[j]], rows_v.at[b], gsems[b])

    def out_slice(j):
        return out_hbm.at[pl.ds((base_ch + j) * CH, CH)]

    for b in range(NBUF):
        start_gather(b, b)

    def body(g, carry):
        for b in range(NBUF):
            j = g * NBUF + b
            pltpu.make_async_copy(
                table_hbm.at[idx_v.at[j]], rows_v.at[b], gsems[b]
            ).wait()
            pltpu.async_copy(rows_v.at[b], out_slice(j), ssems[b])
            @pl.when(g + 1 < NG)
            def _():
                pltpu.make_async_copy(rows_v.at[b], out_slice(j), ssems[b]).wait()
                start_gather(j + NBUF, b)
        return carry

    lax.fori_loop(0, NG, body, 0)

    for b in range(NBUF):
        j = (NG - 1) * NBUF + b
        pltpu.make_async_copy(rows_v.at[b], out_slice(j), ssems[b]).wait()


def kernel(tok_ids, table):
    idx = tok_ids.reshape(NW * NCH, CH).astype(jnp.int32)
    out = _emb(idx, table)
    return out.reshape(B, L, E)
